# baseline (speedup 1.0000x reference)
import functools

import jax
import jax.numpy as jnp
from jax.experimental import pallas as pl
from jax.experimental.pallas import tpu as pltpu

NUM_CLASSES = 3
N = 200 * 176
K = 20000
K_PAD = 20480
PRE_MAX = 500
POST_MAX = 83
IOU_TH = 0.2
SCORE_TH = 0.1
PCR_LO = (-61.2, -61.2, -10.0)
PCR_HI = (61.2, 61.2, 10.0)


def _mlp_body(x_ref, w1_ref, b1_ref, w2_ref, b2_ref, w3_ref, b3_ref,
              w4_ref, b4_ref, out_ref):
    x = x_ref[0]
    h = jnp.maximum(jnp.dot(w1_ref[0], x, preferred_element_type=jnp.float32)
                    + b1_ref[0], 0.0)
    h = jnp.maximum(jnp.dot(w2_ref[0], h, preferred_element_type=jnp.float32)
                    + b2_ref[0], 0.0)
    h = jnp.maximum(jnp.dot(w3_ref[0], h, preferred_element_type=jnp.float32)
                    + b3_ref[0], 0.0)
    out_ref[0] = (jnp.dot(w4_ref[0], h, preferred_element_type=jnp.float32)
                  + b4_ref[0])


def _mlp(x, W1, b1, W2, b2, W3, b3, W4, b4):
    spec = lambda shape: pl.BlockSpec((1,) + shape, lambda c: (c,) + (0,) * len(shape))
    out = pl.pallas_call(
        _mlp_body,
        grid=(NUM_CLASSES,),
        in_specs=[
            spec((4, K_PAD)),
            spec((18, 4)), spec((18, 1)),
            spec((36, 18)), spec((36, 1)),
            spec((36, 36)), spec((36, 1)),
            spec((1, 36)), spec((1, 1)),
        ],
        out_specs=pl.BlockSpec((1, 1, K_PAD), lambda c: (c, 0, 0)),
        out_shape=jax.ShapeDtypeStruct((NUM_CLASSES, 1, K_PAD), jnp.float32),
    )(x, W1, b1[..., None], W2, b2[..., None], W3, b3[..., None],
      W4, b4[..., None])
    return out.reshape(NUM_CLASSES, K_PAD)


def _bev_nms(boxes, scores):
    cx, cy, dx, dy = boxes[:, 0], boxes[:, 1], boxes[:, 3], boxes[:, 4]
    x1 = cx - dx * 0.5
    x2 = cx + dx * 0.5
    y1 = cy - dy * 0.5
    y2 = cy + dy * 0.5
    area = dx * dy
    iw = jnp.clip(jnp.minimum(x2[:, None], x2[None, :]) - jnp.maximum(x1[:, None], x1[None, :]), 0.0)
    ih = jnp.clip(jnp.minimum(y2[:, None], y2[None, :]) - jnp.maximum(y1[:, None], y1[None, :]), 0.0)
    inter = iw * ih
    iou = inter / (area[:, None] + area[None, :] - inter + 1e-6)
    valid = scores > 0.0
    P = boxes.shape[0]
    idx = jnp.arange(P)

    def body(i, keep):
        cur = keep[i] & valid[i]
        sup = (iou[i] > IOU_TH) & (idx > i) & cur
        return keep & (~sup)

    return jax.lax.fori_loop(0, P, body, valid)


def kernel(fusion_input, idx_row, idx_col, box_preds, W1, b1, W2, b2, W3, b3, W4, b4):
    x = fusion_input.reshape(NUM_CLASSES, 4, K)
    x = jnp.pad(x, ((0, 0), (0, 0), (0, K_PAD - K)))
    h_all = _mlp(x, W1, b1, W2, b2, W3, b3, W4, b4)[:, :K]

    pcr_lo = jnp.array(PCR_LO, dtype=jnp.float32)
    pcr_hi = jnp.array(PCR_HI, dtype=jnp.float32)
    col_sel = jnp.array([0, 1, 2, 3, 4, 5, 8])
    outs = []
    for t in range(NUM_CLASSES):
        h = h_all[t]
        out1 = jnp.full((100, N), -9999.0, dtype=jnp.float32)
        out1 = out1.at[idx_row[t], idx_col[t]].set(h)
        fused = jnp.max(out1, axis=0)
        hm = jax.nn.sigmoid(fused)
        bp = jax.lax.dynamic_slice_in_dim(box_preds, t * N, N, axis=0)
        dist = jnp.all(bp[:, :3] >= pcr_lo, axis=1) & jnp.all(bp[:, :3] <= pcr_hi, axis=1)
        mask = dist & (hm > SCORE_TH)
        masked = jnp.where(mask, hm, -1.0)
        top_s, top_i = jax.lax.top_k(masked, PRE_MAX)
        tb = bp[top_i]
        keep = _bev_nms(tb[:, col_sel], top_s)
        keep_s = jnp.where(keep, top_s, -1e9)
        sel_s, sel_i = jax.lax.top_k(keep_s, POST_MAX)
        sel_b = tb[sel_i]
        outs.append(jnp.concatenate([sel_b, sel_s[:, None]], axis=1))
    return jnp.stack(outs)


# speedup vs baseline: 5.8986x; 5.8986x over previous
import functools

import jax
import jax.numpy as jnp
from jax.experimental import pallas as pl
from jax.experimental.pallas import tpu as pltpu
from jax.experimental.pallas import tpu_sc as plsc

NUM_CLASSES = 3
N = 200 * 176
K = 20000
K_PAD = 20480
PRE_MAX = 500
POST_MAX = 83
IOU_TH = 0.2
SCORE_TH = 0.1
PCR_LO = (-61.2, -61.2, -10.0)
PCR_HI = (61.2, 61.2, 10.0)


def _mlp_body(x_ref, w1_ref, b1_ref, w2_ref, b2_ref, w3_ref, b3_ref,
              w4_ref, b4_ref, out_ref):
    x = x_ref[0]
    h = jnp.maximum(jnp.dot(w1_ref[0], x, preferred_element_type=jnp.float32)
                    + b1_ref[0], 0.0)
    h = jnp.maximum(jnp.dot(w2_ref[0], h, preferred_element_type=jnp.float32)
                    + b2_ref[0], 0.0)
    h = jnp.maximum(jnp.dot(w3_ref[0], h, preferred_element_type=jnp.float32)
                    + b3_ref[0], 0.0)
    out_ref[0] = (jnp.dot(w4_ref[0], h, preferred_element_type=jnp.float32)
                  + b4_ref[0])


def _mlp(x, W1, b1, W2, b2, W3, b3, W4, b4):
    spec = lambda shape: pl.BlockSpec((1,) + shape, lambda c: (c,) + (0,) * len(shape))
    out = pl.pallas_call(
        _mlp_body,
        grid=(NUM_CLASSES,),
        in_specs=[
            spec((4, K_PAD)),
            spec((18, 4)), spec((18, 1)),
            spec((36, 18)), spec((36, 1)),
            spec((36, 36)), spec((36, 1)),
            spec((1, 36)), spec((1, 1)),
        ],
        out_specs=pl.BlockSpec((1, 1, K_PAD), lambda c: (c, 0, 0)),
        out_shape=jax.ShapeDtypeStruct((NUM_CLASSES, 1, K_PAD), jnp.float32),
    )(x, W1, b1[..., None], W2, b2[..., None], W3, b3[..., None],
      W4, b4[..., None])
    return out.reshape(NUM_CLASSES, K_PAD)


N_PAD = 35328
CPS = 1104
NW = 32
GRID_W = 100 * CPS
CH = 2000
NCH = K // CH
SENT = jnp.int32(0x7FFFFFFF)


def _scatter_fused_sc(idx_row, idx_col, h):
    mesh = plsc.VectorSubcoreMesh(core_axis_name="c", subcore_axis_name="s")

    @functools.partial(
        pl.kernel, mesh=mesh,
        compiler_params=pltpu.CompilerParams(needs_layout_passes=False),
        out_type=jax.ShapeDtypeStruct((NUM_CLASSES * N_PAD,), jnp.float32),
        scratch_types=[
            pltpu.VMEM((GRID_W,), jnp.int32),
            pltpu.VMEM((CPS,), jnp.float32),
            pltpu.VMEM((CH,), jnp.int32),
            pltpu.VMEM((CH,), jnp.int32),
            pltpu.VMEM((CH,), jnp.float32),
            pltpu.VMEM((16,), jnp.int32),
            pltpu.VMEM((16,), jnp.float32),
        ],
    )
    def sc_kernel(row_hbm, col_hbm, h_hbm, out_hbm, grid_v, fused_v, rbuf, qbuf, vbuf, s16i, s16f):
        wid = jax.lax.axis_index("s") * 2 + jax.lax.axis_index("c")
        base = wid * CPS
        lane = jax.lax.iota(jnp.int32, 16)

        def memset_body(i, _):
            grid_v[pl.ds(i * 16, 16)] = jnp.zeros((16,), jnp.int32)
            return 0
        jax.lax.fori_loop(0, GRID_W // 16, memset_body, 0)

        for c in range(NUM_CLASSES):
            tag = jnp.int32((c + 1) << 15)

            def chunkA(j, _):
                off = j * CH
                pltpu.sync_copy(row_hbm.at[pl.ds(c * K + off, CH)], rbuf)
                pltpu.sync_copy(col_hbm.at[pl.ds(c * K + off, CH)], qbuf)

                def bodyA(i, _):
                    r = rbuf[pl.ds(i * 16, 16)]
                    q = qbuf[pl.ds(i * 16, 16)]
                    qr = q - base
                    m = (qr >= 0) & (qr < CPS)
                    a = r * CPS + qr
                    packed = jnp.where(m, a * 16 + lane, SENT)
                    sp = jax.lax.sort(packed)
                    sa = jax.lax.shift_right_logical(sp, 4)
                    sl = jnp.bitwise_and(sp, 15)
                    kk = off + i * 16 + sl
                    pval = tag | kk
                    s16i[...] = sa
                    nx = plsc.load_gather(s16i, [jnp.minimum(lane + 1, 15)])
                    wm = ((sa != nx) | (lane == 15)) & (sp != SENT)
                    a_safe = jnp.where(wm, sa, 0)
                    old = plsc.load_gather(grid_v, [a_safe], mask=wm)
                    plsc.store_scatter(grid_v, [a_safe],
                                       jnp.maximum(old, pval), mask=wm)
                    return 0
                jax.lax.fori_loop(0, CH // 16, bodyA, 0)
                return 0
            jax.lax.fori_loop(0, NCH, chunkA, 0)

            def fused_init(i, _):
                fused_v[pl.ds(i * 16, 16)] = jnp.full((16,), -9999.0, jnp.float32)
                return 0
            jax.lax.fori_loop(0, CPS // 16, fused_init, 0)

            def chunkB(j, _):
                off = j * CH
                pltpu.sync_copy(row_hbm.at[pl.ds(c * K + off, CH)], rbuf)
                pltpu.sync_copy(col_hbm.at[pl.ds(c * K + off, CH)], qbuf)
                pltpu.sync_copy(h_hbm.at[pl.ds(c * K_PAD + off, CH)], vbuf)

                def bodyB(i, _):
                    r = rbuf[pl.ds(i * 16, 16)]
                    q = qbuf[pl.ds(i * 16, 16)]
                    v = vbuf[pl.ds(i * 16, 16)]
                    qr = q - base
                    m = (qr >= 0) & (qr < CPS)
                    a = r * CPS + qr
                    kk = off + i * 16 + lane
                    pval = tag | kk
                    a_safe0 = jnp.where(m, a, 0)
                    w = plsc.load_gather(grid_v, [a_safe0], mask=m)
                    alive = m & (w == pval)
                    packed = jnp.where(alive, qr * 16 + lane, SENT)
                    sp, sv = plsc.sort_key_val(packed, v)
                    scol = jax.lax.shift_right_logical(sp, 4)
                    s16i[...] = scol
                    for d in (1, 2, 4, 8):
                        idxb = jnp.maximum(lane - d, 0)
                        s16f[...] = sv
                        pc = plsc.load_gather(s16i, [idxb])
                        pv = plsc.load_gather(s16f, [idxb])
                        same = (pc == scol) & (lane >= d)
                        sv = jnp.where(same, jnp.maximum(sv, pv), sv)
                    nxc = plsc.load_gather(s16i, [jnp.minimum(lane + 1, 15)])
                    wm = ((scol != nxc) | (lane == 15)) & (sp != SENT)
                    c_safe = jnp.where(wm, scol, 0)
                    oldf = plsc.load_gather(fused_v, [c_safe], mask=wm)
                    plsc.store_scatter(fused_v, [c_safe],
                                       jnp.maximum(oldf, sv), mask=wm)
                    return 0
                jax.lax.fori_loop(0, CH // 16, bodyB, 0)
                return 0
            jax.lax.fori_loop(0, NCH, chunkB, 0)

            pltpu.sync_copy(fused_v, out_hbm.at[pl.ds(c * N_PAD + base, CPS)])

    return sc_kernel(idx_row, idx_col, h)


P = 512
OUT_P = 96


def _nms_body(tb_ref, br_ref, out_ref, iou_ref):
    xr = br_ref[0, 0:1, :]
    yr = br_ref[0, 1:2, :]
    dxr = br_ref[0, 3:4, :]
    dyr = br_ref[0, 4:5, :]
    sr = br_ref[0, 9:10, :]
    xc = tb_ref[0, :, 0:1]
    yc = tb_ref[0, :, 1:2]
    dxc = tb_ref[0, :, 3:4]
    dyc = tb_ref[0, :, 4:5]

    x1c, x2c = xc - dxc * 0.5, xc + dxc * 0.5
    y1c, y2c = yc - dyc * 0.5, yc + dyc * 0.5
    x1r, x2r = xr - dxr * 0.5, xr + dxr * 0.5
    y1r, y2r = yr - dyr * 0.5, yr + dyr * 0.5
    iw = jnp.maximum(jnp.minimum(x2c, x2r) - jnp.maximum(x1c, x1r), 0.0)
    ih = jnp.maximum(jnp.minimum(y2c, y2r) - jnp.maximum(y1c, y1r), 0.0)
    inter = iw * ih
    iou_ref[...] = inter / (dxc * dyc + dxr * dyr - inter + 1e-6)

    lanes = jax.lax.broadcasted_iota(jnp.int32, (1, P), 1)
    keep0 = jnp.where(sr > 0.0, 1.0, 0.0)

    def body(i, keep):
        m = lanes == i
        cur = jnp.sum(jnp.where(m, keep, 0.0)) > 0.0
        row = iou_ref[pl.ds(i, 1), :]
        sup = (row > IOU_TH) & (lanes > i) & cur
        return jnp.where(sup, 0.0, keep)

    kept_f = jax.lax.fori_loop(0, PRE_MAX, body, keep0)
    keep = kept_f > 0.5
    kept_count = jnp.sum(kept_f)
    tri = (jax.lax.broadcasted_iota(jnp.int32, (P, P), 0)
           <= jax.lax.broadcasted_iota(jnp.int32, (P, P), 1)).astype(jnp.float32)
    csum_k = jnp.dot(kept_f, tri, preferred_element_type=jnp.float32)
    csum_u = jnp.dot(1.0 - kept_f, tri, preferred_element_type=jnp.float32)
    pos = jnp.where(keep, csum_k - 1.0, kept_count + csum_u - 1.0)
    sel = (jnp.broadcast_to(pos, (OUT_P, P))
           == jax.lax.broadcasted_iota(jnp.int32, (OUT_P, P), 0).astype(jnp.float32)
           ).astype(jnp.float32)
    out16 = jnp.dot(sel, tb_ref[0], preferred_element_type=jnp.float32, precision=jax.lax.Precision.HIGHEST)
    jcol = jax.lax.broadcasted_iota(jnp.int32, (OUT_P, 1), 0).astype(jnp.float32)
    out_ref[0] = out16
    out_ref[0, :, 9:10] = jnp.where(jcol < kept_count, out16[:, 9:10], -1e9)


def _nms_topk(tb_ext, br_ext):
    return pl.pallas_call(
        _nms_body,
        grid=(NUM_CLASSES,),
        in_specs=[
            pl.BlockSpec((1, P, 16), lambda c: (c, 0, 0)),
            pl.BlockSpec((1, 16, P), lambda c: (c, 0, 0)),
        ],
        out_specs=pl.BlockSpec((1, OUT_P, 16), lambda c: (c, 0, 0)),
        out_shape=jax.ShapeDtypeStruct((NUM_CLASSES, OUT_P, 16), jnp.float32),
        scratch_shapes=[pltpu.VMEM((P, P), jnp.float32)],
    )(tb_ext, br_ext)


def kernel(fusion_input, idx_row, idx_col, box_preds, W1, b1, W2, b2, W3, b3, W4, b4):
    x = fusion_input.reshape(NUM_CLASSES, 4, K)
    x = jnp.pad(x, ((0, 0), (0, 0), (0, K_PAD - K)))
    h_full = _mlp(x, W1, b1, W2, b2, W3, b3, W4, b4)

    fused_all = _scatter_fused_sc(
        idx_row.reshape(-1).astype(jnp.int32),
        idx_col.reshape(-1).astype(jnp.int32),
        h_full.reshape(-1)).reshape(NUM_CLASSES, N_PAD)[:, :N]

    pcr_lo = jnp.array(PCR_LO, dtype=jnp.float32)
    pcr_hi = jnp.array(PCR_HI, dtype=jnp.float32)
    tbs = []
    for t in range(NUM_CLASSES):
        fused = fused_all[t]
        hm = jax.nn.sigmoid(fused)
        bp = jax.lax.dynamic_slice_in_dim(box_preds, t * N, N, axis=0)
        dist = jnp.all(bp[:, :3] >= pcr_lo, axis=1) & jnp.all(bp[:, :3] <= pcr_hi, axis=1)
        mask = dist & (hm > SCORE_TH)
        masked = jnp.where(mask, hm, -1.0)
        top_s, top_i = jax.lax.top_k(masked, PRE_MAX)
        tb = bp[top_i]
        tb16 = jnp.concatenate(
            [tb, top_s[:, None], jnp.zeros((PRE_MAX, 6), jnp.float32)], axis=1)
        tbs.append(jnp.pad(tb16, ((0, P - PRE_MAX), (0, 0)),
                           constant_values=0.0).at[PRE_MAX:, 9].set(-1e9))
    tb_ext = jnp.stack(tbs)
    br_ext = jnp.transpose(tb_ext, (0, 2, 1))
    out = _nms_topk(tb_ext, br_ext)
    return out[:, :POST_MAX, :10]
